# baseline (speedup 1.0000x reference)
import jax
import jax.numpy as jnp
import numpy as np
from jax.experimental import pallas as pl
from jax.experimental.pallas import tpu as pltpu

_EPS = 0.001
_TOPK = 128
_INT_MIN = np.int32(-2147483648)
_INT_MAX = np.int32(2147483647)


def _ordered_key(x):
    u = jax.lax.bitcast_convert_type(x, jnp.int32)
    return jnp.where(u >= 0, u, jnp.bitwise_xor(jnp.bitwise_not(u), _INT_MIN))


def _select_cutoff_key(key):
    def body(_, lohi):
        lo, hi = lohi
        mid = (lo >> 1) + (hi >> 1) + (lo & hi & 1)
        cnt = jnp.sum((key > mid).astype(jnp.int32))
        take_hi = cnt <= _TOPK
        return (jnp.where(take_hi, lo, mid + 1), jnp.where(take_hi, mid, hi))

    lo, _ = jax.lax.fori_loop(0, 32, body, (_INT_MIN, _INT_MAX))
    return lo


def _fused_body(q_ref, k_ref, v_ref, o_ref, acc_ref, sc_s_ref, sc_c_ref, cut_ref):
    p = pl.program_id(1)
    nl = pl.program_id(2)
    lc = q_ref.shape[1]

    @pl.when(jnp.logical_and(p == 0, nl == 0))
    def _():
        acc_ref[...] = jnp.zeros_like(acc_ref)

    @pl.when(p == 0)
    def _():
        q = q_ref[0]
        acc_ref[...] += jnp.sum(jax.nn.relu(q) + _EPS, axis=0, keepdims=True)

    @pl.when(p == 1)
    def _():
        kp = jax.nn.relu(k_ref[0]) + _EPS
        s = jnp.sum(kp * acc_ref[...], axis=-1, keepdims=True)
        sc_s_ref[pl.ds(nl * lc, lc), :] = s
        rows = lc // 128
        sc_c_ref[pl.ds(nl * rows, rows), :] = s.reshape(rows, 128)

    @pl.when(jnp.logical_and(p == 2, nl == 0))
    def _():
        cut_ref[0] = _select_cutoff_key(_ordered_key(sc_c_ref[...]))

    @pl.when(p == 2)
    def _():
        key_s = _ordered_key(sc_s_ref[pl.ds(nl * lc, lc), :])
        m = (key_s > cut_ref[0]).astype(jnp.float32)
        o_ref[0] = v_ref[0] * m


def kernel(queries, keys, values):
    B, L, D = queries.shape
    lc = 512 if L % 512 == 0 else 256
    nl = L // lc

    grid = (B, 3, nl)
    in_specs = [
        pl.BlockSpec((1, lc, D), lambda b, p, n: (b, jnp.where(p == 0, n, 0), 0)),
        pl.BlockSpec((1, lc, D), lambda b, p, n: (b, jnp.where(p == 1, n, 0), 0)),
        pl.BlockSpec((1, lc, D), lambda b, p, n: (b, jnp.where(p == 2, n, 0), 0)),
    ]
    out_spec = pl.BlockSpec((1, lc, D), lambda b, p, n: (b, jnp.where(p == 2, n, 0), 0))

    return pl.pallas_call(
        _fused_body,
        grid=grid,
        in_specs=in_specs,
        out_specs=out_spec,
        out_shape=jax.ShapeDtypeStruct((B, L, D), jnp.float32),
        scratch_shapes=[
            pltpu.VMEM((1, D), jnp.float32),
            pltpu.VMEM((L, 1), jnp.float32),
            pltpu.VMEM((L // 128, 128), jnp.float32),
            pltpu.SMEM((1,), jnp.int32),
        ],
    )(queries, keys, values)


# speedup vs baseline: 1.1101x; 1.1101x over previous
import jax
import jax.numpy as jnp
import numpy as np
from jax.experimental import pallas as pl
from jax.experimental.pallas import tpu as pltpu

_EPS = 0.001
_TOPK = 128
_INT_MIN = np.int32(-2147483648)
_INT_MAX = np.int32(2147483647)


def _ordered_key(x):
    u = jax.lax.bitcast_convert_type(x, jnp.int32)
    return jnp.where(u >= 0, u, jnp.bitwise_xor(jnp.bitwise_not(u), _INT_MIN))


def _select_cutoff_key(key):
    def body(_, lohi):
        lo, hi = lohi
        mid = (lo >> 1) + (hi >> 1) + (lo & hi & 1)
        cnt = jnp.sum((key > mid).astype(jnp.int32))
        take_hi = cnt <= _TOPK
        return (jnp.where(take_hi, lo, mid + 1), jnp.where(take_hi, mid, hi))

    lo, _ = jax.lax.fori_loop(0, 32, body, (_INT_MIN, _INT_MAX))
    return lo


def _fused_body(q_ref, k_ref, v_ref, o_ref, acc_ref, sc_s_ref, sc_c_ref, cut_ref):
    p = pl.program_id(1)
    nl = pl.program_id(2)
    lc = q_ref.shape[1]

    @pl.when(jnp.logical_and(p == 0, nl == 0))
    def _():
        acc_ref[...] = jnp.zeros_like(acc_ref)

    @pl.when(p == 0)
    def _():
        q = q_ref[0]
        acc_ref[...] += jnp.sum(jax.nn.relu(q) + _EPS, axis=0, keepdims=True)

    @pl.when(p == 1)
    def _():
        kp = jax.nn.relu(k_ref[0]) + _EPS
        s = jnp.sum(kp * acc_ref[...], axis=-1, keepdims=True)
        sc_s_ref[pl.ds(nl * lc, lc), :] = s
        rows = lc // 128
        sc_c_ref[pl.ds(nl * rows, rows), :] = s.reshape(rows, 128)

    @pl.when(jnp.logical_and(p == 2, nl == 0))
    def _():
        cut_ref[0] = _select_cutoff_key(_ordered_key(sc_c_ref[...]))

    @pl.when(p == 2)
    def _():
        key_s = _ordered_key(sc_s_ref[pl.ds(nl * lc, lc), :])
        m = (key_s > cut_ref[0]).astype(jnp.float32)
        o_ref[0] = v_ref[0] * m


def kernel(queries, keys, values):
    B, L, D = queries.shape
    lc = 1024 if L % 1024 == 0 else 256
    nl = L // lc

    grid = (B, 3, nl)
    in_specs = [
        pl.BlockSpec((1, lc, D), lambda b, p, n: (b, jnp.where(p == 0, n, 0), 0)),
        pl.BlockSpec((1, lc, D), lambda b, p, n: (b, jnp.where(p == 1, n, 0), 0)),
        pl.BlockSpec((1, lc, D), lambda b, p, n: (b, jnp.where(p == 2, n, 0), 0)),
    ]
    out_spec = pl.BlockSpec((1, lc, D), lambda b, p, n: (b, jnp.where(p == 2, n, 0), 0))

    return pl.pallas_call(
        _fused_body,
        grid=grid,
        in_specs=in_specs,
        out_specs=out_spec,
        out_shape=jax.ShapeDtypeStruct((B, L, D), jnp.float32),
        scratch_shapes=[
            pltpu.VMEM((1, D), jnp.float32),
            pltpu.VMEM((L, 1), jnp.float32),
            pltpu.VMEM((L // 128, 128), jnp.float32),
            pltpu.SMEM((1,), jnp.int32),
        ],
    )(queries, keys, values)


# speedup vs baseline: 1.4520x; 1.3079x over previous
import jax
import jax.numpy as jnp
import numpy as np
from jax.experimental import pallas as pl
from jax.experimental.pallas import tpu as pltpu

_EPS = 0.001
_TOPK = 128
_INT_MIN = np.int32(-2147483648)
_INT_MAX = np.int32(2147483647)


def _ordered_key(x):
    u = jax.lax.bitcast_convert_type(x, jnp.int32)
    return jnp.where(u >= 0, u, jnp.bitwise_xor(jnp.bitwise_not(u), _INT_MIN))


def _select_cutoff_key(key):
    def body(_, lohi):
        lo, hi = lohi
        mid = (lo >> 1) + (hi >> 1) + (lo & hi & 1)
        cnt = jnp.sum((key > mid).astype(jnp.int32))
        take_hi = cnt <= _TOPK
        return (jnp.where(take_hi, lo, mid + 1), jnp.where(take_hi, mid, hi))

    lo, _ = jax.lax.fori_loop(0, 32, body, (_INT_MIN, _INT_MAX))
    return lo


def _make_body(B):
    def _body(q_ref, k_ref, v_ref, o_ref, acc_ref, sc_s_ref, sc_c_ref, cut_ref):
        s = pl.program_id(0)
        n = pl.program_id(1)
        lc = q_ref.shape[1]
        rows = lc // 128
        par = s % 2
        par1 = (s + 1) % 2

        @pl.when(jnp.logical_and(s < B, n == 0))
        def _():
            acc_ref[pl.ds(par, 1)] = jnp.zeros_like(acc_ref[pl.ds(par, 1)])

        @pl.when(s < B)
        def _():
            q = q_ref[0]
            acc_ref[pl.ds(par, 1), 0] += jnp.sum(
                jax.nn.relu(q) + _EPS, axis=0, keepdims=True)

        @pl.when(jnp.logical_and(s >= 1, s <= B))
        def _():
            kp = jax.nn.relu(k_ref[0]) + _EPS
            acc = acc_ref[pl.ds(par1, 1), 0]
            sv = jnp.sum(kp * acc, axis=-1, keepdims=True)
            sc_s_ref[pl.ds(par1, 1), pl.ds(n * lc, lc), :] = sv[None]
            sc_c_ref[pl.ds(par1, 1), pl.ds(n * rows, rows), :] = (
                sv.reshape(rows, 128)[None])

        @pl.when(jnp.logical_and(s >= 2, n == 0))
        def _():
            key = _ordered_key(sc_c_ref[pl.ds(par, 1)][0])
            cut_ref[0] = _select_cutoff_key(key)

        @pl.when(s >= 2)
        def _():
            key_s = _ordered_key(sc_s_ref[pl.ds(par, 1), pl.ds(n * lc, lc), :][0])
            m = (key_s > cut_ref[0]).astype(jnp.float32)
            o_ref[0] = v_ref[0] * m

    return _body


def kernel(queries, keys, values):
    B, L, D = queries.shape
    lc = 1024 if L % 1024 == 0 else 256
    NL = L // lc

    def q_idx(s, n):
        return (jnp.minimum(s, B - 1), jnp.where(s < B, n, NL - 1), 0)

    def k_idx(s, n):
        b = jnp.clip(s - 1, 0, B - 1)
        c = jnp.where(s < 1, 0, jnp.where(s <= B, n, NL - 1))
        return (b, c, 0)

    def v_idx(s, n):
        return (jnp.clip(s - 2, 0, B - 1), jnp.where(s >= 2, n, 0), 0)

    return pl.pallas_call(
        _make_body(B),
        grid=(B + 2, NL),
        in_specs=[
            pl.BlockSpec((1, lc, D), q_idx),
            pl.BlockSpec((1, lc, D), k_idx),
            pl.BlockSpec((1, lc, D), v_idx),
        ],
        out_specs=pl.BlockSpec((1, lc, D), v_idx),
        out_shape=jax.ShapeDtypeStruct((B, L, D), jnp.float32),
        scratch_shapes=[
            pltpu.VMEM((2, 1, D), jnp.float32),
            pltpu.VMEM((2, L, 1), jnp.float32),
            pltpu.VMEM((2, L // 128, 128), jnp.float32),
            pltpu.SMEM((1,), jnp.int32),
        ],
    )(queries, keys, values)


# speedup vs baseline: 1.4523x; 1.0003x over previous
import jax
import jax.numpy as jnp
import numpy as np
from jax.experimental import pallas as pl
from jax.experimental.pallas import tpu as pltpu

_EPS = 0.001
_TOPK = 128
_INT_MIN = np.int32(-2147483648)
_INT_MAX = np.int32(2147483647)


def _ordered_key(x):
    u = jax.lax.bitcast_convert_type(x, jnp.int32)
    return jnp.where(u >= 0, u, jnp.bitwise_xor(jnp.bitwise_not(u), _INT_MIN))


def _select_cutoff_key(key):
    def body(_, lohi):
        lo, hi = lohi
        mid = (lo >> 1) + (hi >> 1) + (lo & hi & 1)
        cnt = jnp.sum((key > mid).astype(jnp.int32))
        take_hi = cnt <= _TOPK
        return (jnp.where(take_hi, lo, mid + 1), jnp.where(take_hi, mid, hi))

    lo, _ = jax.lax.fori_loop(0, 32, body, (_INT_MIN, _INT_MAX))
    return lo


def _make_body(B):
    def _body(q_ref, k_ref, v_ref, o_ref, acc_ref, sc_s_ref, sc_c_ref, cut_ref):
        s = pl.program_id(0)
        n = pl.program_id(1)
        lc = q_ref.shape[1]
        rows = lc // 128
        par = s % 2
        par1 = (s + 1) % 2

        for p in (0, 1):
            @pl.when(jnp.logical_and(jnp.logical_and(s < B, n == 0), par == p))
            def _(p=p):
                acc_ref[p] = jnp.zeros_like(acc_ref[p])

            @pl.when(jnp.logical_and(s < B, par == p))
            def _(p=p):
                q = q_ref[0]
                acc_ref[p] += jnp.sum(
                    jax.nn.relu(q) + _EPS, axis=0, keepdims=True)

        for p in (0, 1):
            @pl.when(jnp.logical_and(
                jnp.logical_and(s >= 1, s <= B), par1 == p))
            def _(p=p):
                kp = jax.nn.relu(k_ref[0]) + _EPS
                sv = jnp.sum(kp * acc_ref[p], axis=-1, keepdims=True)
                sc_s_ref[p, pl.ds(n * lc, lc), :] = sv
                sc_c_ref[p, pl.ds(n * rows, rows), :] = sv.reshape(rows, 128)

        for p in (0, 1):
            @pl.when(jnp.logical_and(
                jnp.logical_and(s >= 2, n == 0), par == p))
            def _(p=p):
                key = _ordered_key(sc_c_ref[p])
                cut_ref[0] = _select_cutoff_key(key)

            @pl.when(jnp.logical_and(s >= 2, par == p))
            def _(p=p):
                key_s = _ordered_key(sc_s_ref[p, pl.ds(n * lc, lc), :])
                m = (key_s > cut_ref[0]).astype(jnp.float32)
                o_ref[0] = v_ref[0] * m

    return _body


def kernel(queries, keys, values):
    B, L, D = queries.shape
    lc = 1024 if L % 1024 == 0 else 256
    NL = L // lc

    def q_idx(s, n):
        return (jnp.minimum(s, B - 1), jnp.where(s < B, n, NL - 1), 0)

    def k_idx(s, n):
        b = jnp.clip(s - 1, 0, B - 1)
        c = jnp.where(s < 1, 0, jnp.where(s <= B, n, NL - 1))
        return (b, c, 0)

    def v_idx(s, n):
        return (jnp.clip(s - 2, 0, B - 1), jnp.where(s >= 2, n, 0), 0)

    return pl.pallas_call(
        _make_body(B),
        grid=(B + 2, NL),
        in_specs=[
            pl.BlockSpec((1, lc, D), q_idx),
            pl.BlockSpec((1, lc, D), k_idx),
            pl.BlockSpec((1, lc, D), v_idx),
        ],
        out_specs=pl.BlockSpec((1, lc, D), v_idx),
        out_shape=jax.ShapeDtypeStruct((B, L, D), jnp.float32),
        scratch_shapes=[
            pltpu.VMEM((2, 1, D), jnp.float32),
            pltpu.VMEM((2, L, 1), jnp.float32),
            pltpu.VMEM((2, L // 128, 128), jnp.float32),
            pltpu.SMEM((1,), jnp.int32),
        ],
    )(queries, keys, values)
